# baseline (speedup 1.0000x reference)
import functools

import jax
import jax.numpy as jnp
from jax import lax
from jax.experimental import pallas as pl
from jax.experimental.pallas import tpu as pltpu
from jax.experimental.pallas import tpu_sc as plsc

N_NODES = 10000
N_PAD = 10240
D_FEAT = 128
DH = 64
N_EDGES = 320000
EPT = N_EDGES // 16
CHUNK = 128
NCHUNK = 157
TAIL_PATCH = NCHUNK * CHUNK - EPT
NBUF = 5
LAG = 3
GLAG = 1
NMAIN = 155
TRASH_ROW = N_NODES
ROWS_PER_TILE = N_PAD // 16
STAGE_ROWS = N_NODES // 16


def _make_sc_call():
  mesh = plsc.VectorSubcoreMesh(core_axis_name="c", subcore_axis_name="s")

  @functools.partial(
      pl.kernel,
      mesh=mesh,
      out_type=jax.ShapeDtypeStruct((N_NODES, D_FEAT), jnp.float32),
      compiler_params=pltpu.CompilerParams(use_tc_tiling_on_sc=False),
      scratch_types=[
          pltpu.VMEM((NBUF, 2, CHUNK), jnp.int32),
          pltpu.VMEM((NBUF, CHUNK, DH), jnp.float32),
          pltpu.VMEM_SHARED((N_PAD, DH), jnp.float32),
          pltpu.VMEM_SHARED((N_PAD, DH), jnp.float32),
          pltpu.SemaphoreType.DMA((NBUF,)),
          pltpu.SemaphoreType.DMA((NBUF,)),
          pltpu.SemaphoreType.DMA((NBUF,)),
      ],
  )
  def sc_kernel(feat_hbm, edge_hbm, out_hbm,
                idx_v, rows_v, ftab, acc, isem, gsem, ssem):
    c = lax.axis_index("c")
    s = lax.axis_index("s")

    st0 = s * STAGE_ROWS
    stage = pltpu.async_copy(
        feat_hbm.at[pl.ds(st0, STAGE_ROWS), pl.ds(c * DH, DH)],
        ftab.at[pl.ds(st0, STAGE_ROWS)], gsem.at[0])

    zero16 = jnp.zeros((16,), jnp.float32)

    def _zrow(i, carry):
      for t in range(DH // 16):
        rows_v[0, i, pl.ds(t * 16, 16)] = zero16
      return carry

    lax.fori_loop(0, CHUNK, _zrow, 0)
    row0 = s * ROWS_PER_TILE
    for b in range(ROWS_PER_TILE // CHUNK):
      pltpu.sync_copy(rows_v.at[0], acc.at[pl.ds(row0 + b * CHUNK, CHUNK)])
    stage.wait()
    plsc.subcore_barrier()

    ebase = s * EPT

    def _off(j):
      return ebase + lax.min(j * CHUNK, EPT - CHUNK)

    def _start_idx(j, b):
      pltpu.async_copy(edge_hbm.at[0, pl.ds(_off(j), CHUNK)],
                       idx_v.at[b, 0], isem.at[b])
      pltpu.async_copy(edge_hbm.at[1, pl.ds(_off(j), CHUNK)],
                       idx_v.at[b, 1], isem.at[b])

    def _wait_idx(j, b):
      pltpu.make_async_copy(edge_hbm.at[0, pl.ds(_off(j), CHUNK)],
                            idx_v.at[b, 0], isem.at[b]).wait()
      pltpu.make_async_copy(edge_hbm.at[1, pl.ds(_off(j), CHUNK)],
                            idx_v.at[b, 1], isem.at[b]).wait()

    def _patch_tail(b):
      trash16 = jnp.full((16,), TRASH_ROW, jnp.int32)
      zero16i = jnp.zeros((16,), jnp.int32)
      for t in range(TAIL_PATCH // 16):
        idx_v[b, 0, pl.ds(t * 16, 16)] = zero16i
        idx_v[b, 1, pl.ds(t * 16, 16)] = trash16

    def _start_gather(b):
      pltpu.async_copy(ftab.at[idx_v.at[b, 0]], rows_v.at[b], gsem.at[b])

    def _wait_gather(b):
      pltpu.make_async_copy(
          ftab.at[idx_v.at[b, 0]], rows_v.at[b], gsem.at[b]).wait()

    def _start_scatter(b):
      pltpu.async_copy(rows_v.at[b], acc.at[idx_v.at[b, 1]], ssem.at[b],
                       add=True)

    def _wait_scatter(b):
      pltpu.make_async_copy(
          rows_v.at[b], acc.at[idx_v.at[b, 1]], ssem.at[b]).wait()

    for b in range(NBUF):
      _start_idx(b, b)
    for b in range(GLAG):
      _wait_idx(b, b)
      _start_gather(b)

    def _group(g, carry):
      for b in range(NBUF):
        j = g * NBUF + b
        _wait_gather(b)
        _start_scatter(b)
        jj = j - LAG
        bb = (b - LAG) % NBUF

        @pl.when(jj >= 0)
        def _():
          _wait_scatter(bb)
          ji = jj + NBUF

          @pl.when(ji < NCHUNK)
          def _():
            _start_idx(ji, bb)

        jg = j + GLAG
        bg = (b + GLAG) % NBUF

        @pl.when(jg < NCHUNK)
        def _():
          _wait_idx(jg, bg)

          @pl.when(jg == NCHUNK - 1)
          def _():
            _patch_tail(bg)

          _start_gather(bg)

      return carry

    lax.fori_loop(0, NMAIN // NBUF, _group, 0)
    for j in range(NMAIN, NCHUNK):
      b = j % NBUF
      if j >= NMAIN + GLAG:
        _wait_idx(j, b)
        if j == NCHUNK - 1:
          _patch_tail(b)
        _start_gather(b)
      _wait_gather(b)
      _start_scatter(b)
    for j in range(NMAIN - LAG, NCHUNK):
      _wait_scatter(j % NBUF)
    plsc.subcore_barrier()

    @pl.when(s < 15)
    def _():
      pltpu.sync_copy(
          acc.at[pl.ds(row0, ROWS_PER_TILE)],
          out_hbm.at[pl.ds(row0, ROWS_PER_TILE), pl.ds(c * DH, DH)])

    @pl.when(s == 15)
    def _():
      last = N_NODES - 15 * ROWS_PER_TILE
      pltpu.sync_copy(
          acc.at[pl.ds(15 * ROWS_PER_TILE, last)],
          out_hbm.at[pl.ds(15 * ROWS_PER_TILE, last), pl.ds(c * DH, DH)])

  return sc_kernel


_sc_call = _make_sc_call()


def kernel(feat, edge_index):
  return _sc_call(feat, edge_index.astype(jnp.int32))


# speedup vs baseline: 1.1317x; 1.1317x over previous
import functools

import jax
import jax.numpy as jnp
from jax import lax
from jax.experimental import pallas as pl
from jax.experimental.pallas import tpu as pltpu
from jax.experimental.pallas import tpu_sc as plsc

N_NODES = 10000
N_PAD = 10240
N_ACC = 10048
D_FEAT = 128
DH = 64
N_EDGES = 320000
EPT = N_EDGES // 16
CHUNK = 128
NCHUNK = 157
TAIL_PATCH = NCHUNK * CHUNK - EPT
NBUF = 6
LAG = 3
GLAG = 2
NMAIN = 150
TRASH_ROW = N_NODES
ROWS_PER_TILE = N_PAD // 16
STAGE_ROWS = N_NODES // 16


def _make_sc_call():
  mesh = plsc.VectorSubcoreMesh(core_axis_name="c", subcore_axis_name="s")

  @functools.partial(
      pl.kernel,
      mesh=mesh,
      out_type=jax.ShapeDtypeStruct((N_NODES, D_FEAT), jnp.float32),
      compiler_params=pltpu.CompilerParams(use_tc_tiling_on_sc=False),
      scratch_types=[
          pltpu.VMEM((NBUF, 2, CHUNK), jnp.int32),
          pltpu.VMEM((NBUF, CHUNK, DH), jnp.float32),
          pltpu.VMEM_SHARED((N_NODES, DH), jnp.float32),
          pltpu.VMEM_SHARED((N_ACC, DH), jnp.float32),
          pltpu.SemaphoreType.DMA((NBUF,)),
          pltpu.SemaphoreType.DMA((NBUF,)),
          pltpu.SemaphoreType.DMA((NBUF,)),
      ],
  )
  def sc_kernel(feat_hbm, edge_hbm, out_hbm,
                idx_v, rows_v, ftab, acc, isem, gsem, ssem):
    c = lax.axis_index("c")
    s = lax.axis_index("s")

    st0 = s * STAGE_ROWS
    stage = pltpu.async_copy(
        feat_hbm.at[pl.ds(st0, STAGE_ROWS), pl.ds(c * DH, DH)],
        ftab.at[pl.ds(st0, STAGE_ROWS)], gsem.at[0])

    zero16 = jnp.zeros((16,), jnp.float32)

    def _zrow(i, carry):
      for t in range(DH // 16):
        rows_v[0, i, pl.ds(t * 16, 16)] = zero16
      return carry

    lax.fori_loop(0, CHUNK, _zrow, 0)
    row0 = s * ROWS_PER_TILE
    zrow0 = s * (N_ACC // 16)
    zoff = 0
    for blk in (128, 128, 128, 128, 116):
      pltpu.sync_copy(rows_v.at[0, pl.ds(0, blk)],
                      acc.at[pl.ds(zrow0 + zoff, blk)])
      zoff += blk
    stage.wait()
    plsc.subcore_barrier()

    ebase = s * EPT

    def _off(j):
      return ebase + lax.min(j * CHUNK, EPT - CHUNK)

    def _start_idx(j, b):
      pltpu.async_copy(edge_hbm.at[0, pl.ds(_off(j), CHUNK)],
                       idx_v.at[b, 0], isem.at[b])
      pltpu.async_copy(edge_hbm.at[1, pl.ds(_off(j), CHUNK)],
                       idx_v.at[b, 1], isem.at[b])

    def _wait_idx(j, b):
      pltpu.make_async_copy(edge_hbm.at[0, pl.ds(_off(j), CHUNK)],
                            idx_v.at[b, 0], isem.at[b]).wait()
      pltpu.make_async_copy(edge_hbm.at[1, pl.ds(_off(j), CHUNK)],
                            idx_v.at[b, 1], isem.at[b]).wait()

    def _patch_tail(b):
      trash16 = jnp.full((16,), TRASH_ROW, jnp.int32)
      zero16i = jnp.zeros((16,), jnp.int32)
      for t in range(TAIL_PATCH // 16):
        idx_v[b, 0, pl.ds(t * 16, 16)] = zero16i
        idx_v[b, 1, pl.ds(t * 16, 16)] = trash16

    def _start_gather(b):
      pltpu.async_copy(ftab.at[idx_v.at[b, 0]], rows_v.at[b], gsem.at[b])

    def _wait_gather(b):
      pltpu.make_async_copy(
          ftab.at[idx_v.at[b, 0]], rows_v.at[b], gsem.at[b]).wait()

    def _start_scatter(b):
      pltpu.async_copy(rows_v.at[b], acc.at[idx_v.at[b, 1]], ssem.at[b],
                       add=True)

    def _wait_scatter(b):
      pltpu.make_async_copy(
          rows_v.at[b], acc.at[idx_v.at[b, 1]], ssem.at[b]).wait()

    for b in range(NBUF):
      _start_idx(b, b)
    for b in range(GLAG):
      _wait_idx(b, b)
      _start_gather(b)

    def _group(g, carry):
      for b in range(NBUF):
        j = g * NBUF + b
        _wait_gather(b)
        _start_scatter(b)
        jj = j - LAG
        bb = (b - LAG) % NBUF

        @pl.when(jj >= 0)
        def _():
          _wait_scatter(bb)
          ji = jj + NBUF

          @pl.when(ji < NCHUNK)
          def _():
            _start_idx(ji, bb)

        jg = j + GLAG
        bg = (b + GLAG) % NBUF

        @pl.when(jg < NCHUNK)
        def _():
          _wait_idx(jg, bg)

          @pl.when(jg == NCHUNK - 1)
          def _():
            _patch_tail(bg)

          _start_gather(bg)

      return carry

    lax.fori_loop(0, NMAIN // NBUF, _group, 0)
    for j in range(NMAIN, NCHUNK + LAG):
      b = j % NBUF
      if j < NCHUNK:
        _wait_gather(b)
        _start_scatter(b)
      jj = j - LAG
      if 0 <= jj < NCHUNK:
        _wait_scatter(jj % NBUF)
        ji = jj + NBUF
        if ji < NCHUNK:
          _start_idx(ji, ji % NBUF)
      jg = j + GLAG
      if jg < NCHUNK:
        bg = jg % NBUF
        _wait_idx(jg, bg)
        if jg == NCHUNK - 1:
          _patch_tail(bg)
        _start_gather(bg)
    plsc.subcore_barrier()

    @pl.when(s < 15)
    def _():
      pltpu.sync_copy(
          acc.at[pl.ds(row0, ROWS_PER_TILE)],
          out_hbm.at[pl.ds(row0, ROWS_PER_TILE), pl.ds(c * DH, DH)])

    @pl.when(s == 15)
    def _():
      last = N_NODES - 15 * ROWS_PER_TILE
      pltpu.sync_copy(
          acc.at[pl.ds(15 * ROWS_PER_TILE, last)],
          out_hbm.at[pl.ds(15 * ROWS_PER_TILE, last), pl.ds(c * DH, DH)])

  return sc_kernel


_sc_call = _make_sc_call()


def kernel(feat, edge_index):
  return _sc_call(feat, edge_index.astype(jnp.int32))


# speedup vs baseline: 1.1356x; 1.0035x over previous
import functools

import jax
import jax.numpy as jnp
from jax import lax
from jax.experimental import pallas as pl
from jax.experimental.pallas import tpu as pltpu
from jax.experimental.pallas import tpu_sc as plsc

N_NODES = 10000
N_PAD = 10240
D_FEAT = 128
DH = 64
N_EDGES = 320000
EPT = N_EDGES // 16
CHUNK = 128
NCHUNK = 157
TAIL_PATCH = NCHUNK * CHUNK - EPT
NBUF = 5
LAG = 2
GLAG = 2
NMAIN = 155
TRASH_ROW = N_NODES
ROWS_PER_TILE = N_PAD // 16
STAGE_ROWS = N_NODES // 16


def _make_sc_call():
  mesh = plsc.VectorSubcoreMesh(core_axis_name="c", subcore_axis_name="s")

  @functools.partial(
      pl.kernel,
      mesh=mesh,
      out_type=jax.ShapeDtypeStruct((N_NODES, D_FEAT), jnp.float32),
      compiler_params=pltpu.CompilerParams(use_tc_tiling_on_sc=False),
      scratch_types=[
          pltpu.VMEM((NBUF, 2, CHUNK), jnp.int32),
          pltpu.VMEM((NBUF, CHUNK, DH), jnp.float32),
          pltpu.VMEM_SHARED((N_PAD, DH), jnp.float32),
          pltpu.VMEM_SHARED((N_PAD, DH), jnp.float32),
          pltpu.SemaphoreType.DMA((NBUF,)),
          pltpu.SemaphoreType.DMA((NBUF,)),
          pltpu.SemaphoreType.DMA((NBUF,)),
      ],
  )
  def sc_kernel(feat_hbm, edge_hbm, out_hbm,
                idx_v, rows_v, ftab, acc, isem, gsem, ssem):
    c = lax.axis_index("c")
    s = lax.axis_index("s")

    st0 = s * STAGE_ROWS
    stage = pltpu.async_copy(
        feat_hbm.at[pl.ds(st0, STAGE_ROWS), pl.ds(c * DH, DH)],
        ftab.at[pl.ds(st0, STAGE_ROWS)], gsem.at[0])

    zero16 = jnp.zeros((16,), jnp.float32)

    def _zrow(i, carry):
      for t in range(DH // 16):
        rows_v[0, i, pl.ds(t * 16, 16)] = zero16
      return carry

    lax.fori_loop(0, CHUNK, _zrow, 0)
    row0 = s * ROWS_PER_TILE
    for b in range(ROWS_PER_TILE // CHUNK):
      pltpu.sync_copy(rows_v.at[0], acc.at[pl.ds(row0 + b * CHUNK, CHUNK)])
    stage.wait()
    plsc.subcore_barrier()

    ebase = s * EPT

    def _off(j):
      return ebase + lax.min(j * CHUNK, EPT - CHUNK)

    def _start_idx(j, b):
      pltpu.async_copy(edge_hbm.at[0, pl.ds(_off(j), CHUNK)],
                       idx_v.at[b, 0], isem.at[b])
      pltpu.async_copy(edge_hbm.at[1, pl.ds(_off(j), CHUNK)],
                       idx_v.at[b, 1], isem.at[b])

    def _wait_idx(j, b):
      pltpu.make_async_copy(edge_hbm.at[0, pl.ds(_off(j), CHUNK)],
                            idx_v.at[b, 0], isem.at[b]).wait()
      pltpu.make_async_copy(edge_hbm.at[1, pl.ds(_off(j), CHUNK)],
                            idx_v.at[b, 1], isem.at[b]).wait()

    def _patch_tail(b):
      trash16 = jnp.full((16,), TRASH_ROW, jnp.int32)
      zero16i = jnp.zeros((16,), jnp.int32)
      for t in range(TAIL_PATCH // 16):
        idx_v[b, 0, pl.ds(t * 16, 16)] = zero16i
        idx_v[b, 1, pl.ds(t * 16, 16)] = trash16

    def _start_gather(b):
      pltpu.async_copy(ftab.at[idx_v.at[b, 0]], rows_v.at[b], gsem.at[b])

    def _wait_gather(b):
      pltpu.make_async_copy(
          ftab.at[idx_v.at[b, 0]], rows_v.at[b], gsem.at[b]).wait()

    def _start_scatter(b):
      pltpu.async_copy(rows_v.at[b], acc.at[idx_v.at[b, 1]], ssem.at[b],
                       add=True)

    def _wait_scatter(b):
      pltpu.make_async_copy(
          rows_v.at[b], acc.at[idx_v.at[b, 1]], ssem.at[b]).wait()

    for b in range(NBUF):
      _start_idx(b, b)
    for b in range(GLAG):
      _wait_idx(b, b)
      _start_gather(b)

    def _group(g, carry):
      for b in range(NBUF):
        j = g * NBUF + b
        _wait_gather(b)
        _start_scatter(b)
        jj = j - LAG
        bb = (b - LAG) % NBUF

        @pl.when(jj >= 0)
        def _():
          _wait_scatter(bb)
          ji = jj + NBUF

          @pl.when(ji < NCHUNK)
          def _():
            _start_idx(ji, bb)

        jg = j + GLAG
        bg = (b + GLAG) % NBUF

        @pl.when(jg < NCHUNK)
        def _():
          _wait_idx(jg, bg)

          @pl.when(jg == NCHUNK - 1)
          def _():
            _patch_tail(bg)

          _start_gather(bg)

      return carry

    lax.fori_loop(0, NMAIN // NBUF, _group, 0)
    for j in range(NMAIN, NCHUNK):
      _wait_gather(j % NBUF)
      _start_scatter(j % NBUF)
    for j in range(NMAIN - LAG, NCHUNK):
      _wait_scatter(j % NBUF)
    plsc.subcore_barrier()

    @pl.when(s < 15)
    def _():
      pltpu.sync_copy(
          acc.at[pl.ds(row0, ROWS_PER_TILE)],
          out_hbm.at[pl.ds(row0, ROWS_PER_TILE), pl.ds(c * DH, DH)])

    @pl.when(s == 15)
    def _():
      last = N_NODES - 15 * ROWS_PER_TILE
      pltpu.sync_copy(
          acc.at[pl.ds(15 * ROWS_PER_TILE, last)],
          out_hbm.at[pl.ds(15 * ROWS_PER_TILE, last), pl.ds(c * DH, DH)])

  return sc_kernel


_sc_call = _make_sc_call()


def kernel(feat, edge_index):
  return _sc_call(feat, edge_index.astype(jnp.int32))
